# baseline (speedup 1.0000x reference)
import jax
import jax.numpy as jnp
from jax.experimental import pallas as pl

_DIM = 1000
_B = 64
_LSPLIT = 2
_LW = 512


def _onehot_body(idx_ref, out_ref):
    j = pl.program_id(1)
    n1 = idx_ref.shape[1]
    idx = idx_ref[...]
    iota = jax.lax.broadcasted_iota(jnp.int32, (_B, n1, _LW), 2) + j * _LW
    out_ref[...] = (iota == idx[:, :, None]).astype(jnp.float32)


def kernel(tensor):
    n0, n1 = tensor.shape
    idx = tensor.astype(jnp.int32)
    return pl.pallas_call(
        _onehot_body,
        grid=(n0 // _B, _LSPLIT),
        in_specs=[pl.BlockSpec((_B, n1), lambda i, j: (i, 0))],
        out_specs=pl.BlockSpec((_B, n1, _LW), lambda i, j: (i, 0, j)),
        out_shape=jax.ShapeDtypeStruct((n0, n1, _DIM), jnp.float32),
    )(idx)
